# Initial kernel scaffold; baseline (speedup 1.0000x reference)
import jax
import jax.numpy as jnp
from jax.experimental import pallas as pl


def kernel(Q, K, V, Wq, bq, Wk, bk, Wv, bv, Wo, bo):
    raise NotImplementedError("write your pallas kernel here")


# speedup vs baseline: 1.0657x; 1.0657x over previous
import functools
import math

import jax
import jax.numpy as jnp
from jax import lax
from jax.experimental import pallas as pl
from jax.experimental.pallas import tpu as pltpu
from jax.experimental.pallas import tpu_sc as plsc

H = 12
DH = 64
DM = 768
NTOP = 45
NSEL = 48
S = H * NSEL
HI = lax.Precision.HIGHEST


def _prep_kernel(Kg_ref, WkT_ref, bk_ref, out_ref):
    out_ref[0] = jnp.dot(Kg_ref[0], WkT_ref[...], precision=None) + bk_ref[...]


def _score_kernel(Qb_ref, WqT_ref, bq_ref, Ksp_ref, Qp_ref, M_ref):
    Lt = Qb_ref.shape[1]
    Qp = jnp.dot(Qb_ref[0], WqT_ref[...], precision=None) + bq_ref[...]
    Qp_ref[0] = Qp
    col = lax.broadcasted_iota(jnp.int32, (Lt, NSEL), 1)
    for h in range(H):
        Ks_h = Ksp_ref[0][:, h * DH:(h + 1) * DH]
        QKs_h = lax.dot_general(
            Qp[:, h * DH:(h + 1) * DH], Ks_h,
            (((1,), (1,)), ((), ())), precision=None)
        mx = jnp.max(jnp.where(col < NTOP, QKs_h, -jnp.inf), axis=1)
        sm = jnp.sum(jnp.where(col < NTOP, QKs_h, 0.0), axis=1)
        M_ref[0, :, h:h + 1] = (mx - sm / NTOP)[:, None]


def _select_kernel(M_ref, idx_ref, v_scr):
    L = M_ref.shape[1]
    nr = L // 128
    v_scr[...] = M_ref[0].T.reshape(H, nr, 128)
    r_io = lax.broadcasted_iota(jnp.int32, (H, nr, 128), 1)
    c_io = lax.broadcasted_iota(jnp.int32, (H, nr, 128), 2)
    lin = r_io * 128 + c_io

    def body(u, carry):
        val = v_scr[...]
        gmax = jnp.max(jnp.max(val, axis=2), axis=1)
        cand = jnp.where(val == gmax[:, None, None], lin, jnp.int32(2 ** 30))
        sel = jnp.min(jnp.min(cand, axis=2), axis=1)
        idx_ref[0, pl.ds(u, 1), :] = sel[None, :]
        v_scr[...] = jnp.where(lin == sel[:, None, None], -3e38, val)
        return carry

    lax.fori_loop(0, NSEL, body, 0)


def _attn_kernel(Kb_ref, QG_ref, WkT_ref, w_ref, pt_ref, qrt_scr, st_scr):
    scale = 1.0 / math.sqrt(DH)
    for h in range(H):
        o = (h % 2) * DH
        QGhT = QG_ref[0, h][:, o:o + DH].T
        qrt_scr[:, h * NSEL:(h + 1) * NSEL] = jnp.dot(
            WkT_ref[:, h * DH:(h + 1) * DH], QGhT, precision=HI)
    L = Kb_ref.shape[1]
    C = 512
    mx = jnp.full((1, S), -jnp.inf, jnp.float32)
    for i in range(L // C):
        stc = jnp.dot(Kb_ref[0, i * C:(i + 1) * C], qrt_scr[...],
                      precision=None) * scale
        st_scr[i * C:(i + 1) * C] = stc
        mx = jnp.maximum(mx, jnp.max(stc, axis=0, keepdims=True))
    den = jnp.zeros((1, S), jnp.float32)
    for i in range(L // C):
        e = jnp.exp(st_scr[i * C:(i + 1) * C] - mx)
        st_scr[i * C:(i + 1) * C] = e
        den = den + jnp.sum(e, axis=0, keepdims=True)
    for i in range(L // C):
        pt_ref[0, i * C:(i + 1) * C] = jnp.dot(
            st_scr[i * C:(i + 1) * C] / den, w_ref[...], precision=None)


def _out_kernel(Vb_ref, pt_ref, WvT_ref, bvr_ref, WoP_ref, bo_ref, out_ref):
    P = pt_ref[0].T
    Ub = jnp.dot(P, Vb_ref[0], precision=None)
    acc = bo_ref[...]
    for h in range(H):
        ctx_h = (jnp.dot(Ub[h:h + 1, :], WvT_ref[:, h * DH:(h + 1) * DH],
                         precision=HI) + bvr_ref[h:h + 1, :])
        acc = acc + jnp.dot(ctx_h, WoP_ref[h], precision=HI)
    out_ref[...] = acc[None]


def kernel(Q, K, V, Wq, bq, Wk, bk, Wv, bv, Wo, bo):
    B, L, _ = Q.shape
    nl = 2
    Lt = L // nl

    idx_s = jax.random.randint(jax.random.key(42), (NTOP,), 0, L)
    Kg = jnp.pad(jnp.take(K, idx_s, axis=1), ((0, 0), (0, NSEL - NTOP), (0, 0)))
    WqT, WkT, WvT = Wq.T, Wk.T, Wv.T
    bvr = bv.reshape(H, DH)
    WoP = Wo.T.reshape(DH, H, DM).transpose(1, 0, 2)
    r = jnp.arange(S)
    w = (((r[:, None] // NSEL) == jnp.arange(H)[None, :])
         & ((r[:, None] % NSEL) < NTOP)).astype(jnp.float32) / NTOP

    Ksp = pl.pallas_call(
        _prep_kernel,
        grid=(B,),
        in_specs=[pl.BlockSpec((1, NSEL, DM), lambda b: (b, 0, 0)),
                  pl.BlockSpec((DM, DM), lambda b: (0, 0)),
                  pl.BlockSpec((1, DM), lambda b: (0, 0))],
        out_specs=pl.BlockSpec((1, NSEL, DM), lambda b: (b, 0, 0)),
        out_shape=jax.ShapeDtypeStruct((B, NSEL, DM), jnp.float32),
    )(Kg, WkT, bk[None])

    Qp, M = pl.pallas_call(
        _score_kernel,
        grid=(B, nl),
        in_specs=[pl.BlockSpec((1, Lt, DM), lambda b, i: (b, i, 0)),
                  pl.BlockSpec((DM, DM), lambda b, i: (0, 0)),
                  pl.BlockSpec((1, DM), lambda b, i: (0, 0)),
                  pl.BlockSpec((1, NSEL, DM), lambda b, i: (b, 0, 0))],
        out_specs=[pl.BlockSpec((1, Lt, DM), lambda b, i: (b, i, 0)),
                   pl.BlockSpec((1, Lt, H), lambda b, i: (b, i, 0))],
        out_shape=[jax.ShapeDtypeStruct((B, L, DM), jnp.float32),
                   jax.ShapeDtypeStruct((B, L, H), jnp.float32)],
    )(Q, WqT, bq[None], Ksp)

    IDX = pl.pallas_call(
        _select_kernel,
        grid=(B,),
        in_specs=[pl.BlockSpec((1, L, H), lambda b: (b, 0, 0))],
        out_specs=pl.BlockSpec((1, NSEL, H), lambda b: (b, 0, 0)),
        out_shape=jax.ShapeDtypeStruct((B, NSEL, H), jnp.int32),
        scratch_shapes=[pltpu.VMEM((H, L // 128, 128), jnp.float32)],
    )(M)

    idxT = IDX.transpose(0, 2, 1)
    nh2 = H // 2
    qp_flat = Qp.reshape(B * L * nh2, 2 * DH)

    mesh = plsc.VectorSubcoreMesh(core_axis_name="c", subcore_axis_name="s")

    @functools.partial(
        pl.kernel, mesh=mesh,
        out_type=jax.ShapeDtypeStruct((B, H, NSEL, 2 * DH), jnp.float32),
        scratch_types=[pltpu.VMEM((NSEL,), jnp.int32),
                       pltpu.VMEM((NSEL,), jnp.int32),
                       pltpu.VMEM((NSEL, 2 * DH), jnp.float32),
                       pltpu.SemaphoreType.DMA])
    def _sc_gather(qpflat_hbm, idxT_hbm, qg_hbm, idx_v, r_v, rows_v, sem):
        wid = lax.axis_index("s") * 2 + lax.axis_index("c")

        @pl.when(wid < B * H)
        def _():
            b = wid // H
            h = wid % H
            pltpu.sync_copy(idxT_hbm.at[b, h], idx_v)
            for j in range(NSEL // 16):
                v = idx_v[pl.ds(j * 16, 16)]
                r_v[pl.ds(j * 16, 16)] = v * nh2 + (b * (L * nh2) + h // 2)
            pltpu.async_copy(qpflat_hbm.at[r_v], rows_v, sem).wait()
            pltpu.sync_copy(rows_v, qg_hbm.at[b, h])

    QG = _sc_gather(qp_flat, idxT)

    PT = pl.pallas_call(
        _attn_kernel,
        grid=(B,),
        in_specs=[pl.BlockSpec((1, L, DM), lambda b: (b, 0, 0)),
                  pl.BlockSpec((1, H, NSEL, 2 * DH), lambda b: (b, 0, 0, 0)),
                  pl.BlockSpec((DM, DM), lambda b: (0, 0)),
                  pl.BlockSpec((S, H), lambda b: (0, 0))],
        out_specs=pl.BlockSpec((1, L, H), lambda b: (b, 0, 0)),
        out_shape=jax.ShapeDtypeStruct((B, L, H), jnp.float32),
        scratch_shapes=[pltpu.VMEM((DM, S), jnp.float32),
                        pltpu.VMEM((L, S), jnp.float32)],
    )(K, QG, WkT, w)

    out = pl.pallas_call(
        _out_kernel,
        grid=(B,),
        in_specs=[pl.BlockSpec((1, L, DM), lambda b: (b, 0, 0)),
                  pl.BlockSpec((1, L, H), lambda b: (b, 0, 0)),
                  pl.BlockSpec((DM, DM), lambda b: (0, 0)),
                  pl.BlockSpec((H, DH), lambda b: (0, 0)),
                  pl.BlockSpec((H, DH, DM), lambda b: (0, 0, 0)),
                  pl.BlockSpec((1, DM), lambda b: (0, 0))],
        out_specs=pl.BlockSpec((1, 1, DM), lambda b: (b, 0, 0)),
        out_shape=jax.ShapeDtypeStruct((B, 1, DM), jnp.float32),
    )(V, PT, WvT, bvr, WoP, bo[None])

    return out.reshape(B, DM)


# speedup vs baseline: 1.2925x; 1.2128x over previous
import functools
import math

import jax
import jax.numpy as jnp
from jax import lax
from jax.experimental import pallas as pl
from jax.experimental.pallas import tpu as pltpu
from jax.experimental.pallas import tpu_sc as plsc

H = 12
DH = 64
DM = 768
NTOP = 45
NSEL = 48
S = H * NSEL
HI = lax.Precision.HIGHEST


def _prep_kernel(Kg_ref, WkT_ref, bk_ref, out_ref):
    out_ref[0] = jnp.dot(Kg_ref[0], WkT_ref[...], precision=None) + bk_ref[...]


def _score_kernel(Qb_ref, WqT_ref, bq_ref, Ksp_ref, Qp_ref, M_ref):
    Lt = Qb_ref.shape[1]
    Qp = jnp.dot(Qb_ref[0], WqT_ref[...], precision=None) + bq_ref[...]
    Qp_ref[0] = Qp
    col = lax.broadcasted_iota(jnp.int32, (Lt, NSEL), 1)
    for h in range(H):
        Ks_h = Ksp_ref[0][:, h * DH:(h + 1) * DH]
        QKs_h = lax.dot_general(
            Qp[:, h * DH:(h + 1) * DH], Ks_h,
            (((1,), (1,)), ((), ())), precision=None)
        mx = jnp.max(jnp.where(col < NTOP, QKs_h, -jnp.inf), axis=1)
        sm = jnp.sum(jnp.where(col < NTOP, QKs_h, 0.0), axis=1)
        M_ref[0, :, h:h + 1] = (mx - sm / NTOP)[:, None]


def _thresh_kernel(M_ref, UL_ref, UB_ref, idx_ref, v_scr):
    L = M_ref.shape[1]
    nr = L // 128
    bits = lax.bitcast_convert_type(M_ref[0], jnp.int32)
    keys = bits ^ ((bits >> 31) & jnp.int32(0x7FFFFFFF))
    v_scr[...] = keys.T.reshape(H, nr, 128)

    def body(it, lh):
        lo, hi = lh
        mid = (lo >> 1) + (hi >> 1) + (lo & hi & 1)
        cmp = (v_scr[...] > mid).astype(jnp.float32)
        cnt = jnp.sum(jnp.sum(cmp, axis=2, keepdims=True),
                      axis=1, keepdims=True)
        ge = cnt >= float(NTOP)
        return jnp.where(ge, mid, lo), jnp.where(ge, hi, mid)

    lo0 = jnp.full((H, 1, 1), jnp.int32(-2 ** 31))
    hi0 = jnp.full((H, 1, 1), jnp.int32(2 ** 31 - 1))
    _, hi = lax.fori_loop(0, 32, body, (lo0, hi0))

    keys3 = v_scr[...]

    def prefix(msk):
        m = msk.astype(jnp.float32).reshape(H * nr, 128)
        pref = jnp.dot(m, UL_ref[...], precision=None)
        tot = pref[:, 127:128].reshape(H, nr)
        bp = jnp.dot(tot, UB_ref[...], precision=None)
        pos = pref.reshape(H, nr, 128) + bp[:, :, None]
        cg = (bp[:, nr - 1] + tot[:, nr - 1]).reshape(H, 1, 1)
        return pos.astype(jnp.int32), cg.astype(jnp.int32)

    gt = keys3 > hi
    eq = keys3 == hi
    pos_gt, cg = prefix(gt)
    pos_eq, _ = prefix(eq)
    big = jnp.int32(1 << 20)
    slot = jnp.where(gt, pos_gt - 1, jnp.where(eq, cg + pos_eq - 1, big))
    slot = jnp.where(slot < NSEL, slot, big)
    r_io = lax.broadcasted_iota(jnp.int32, (H, nr, 128), 1)
    c_io = lax.broadcasted_iota(jnp.int32, (H, nr, 128), 2)
    lin = (r_io * 128 + c_io).astype(jnp.float32)
    for j in range(NSEL):
        v = jnp.where(slot == j, lin, 0.0)
        red = jnp.sum(jnp.sum(v, axis=2, keepdims=True), axis=1)
        idx_ref[0, :, j:j + 1] = red.astype(jnp.int32)


def _attn_kernel(Kb_ref, QG_ref, WkT_ref, w_ref, pt_ref, qrt_scr, st_scr):
    scale = 1.0 / math.sqrt(DH)
    for h in range(H):
        o = (h % 2) * DH
        QGhT = QG_ref[0, h][:, o:o + DH].T
        qrt_scr[:, h * NSEL:(h + 1) * NSEL] = jnp.dot(
            WkT_ref[:, h * DH:(h + 1) * DH], QGhT, precision=HI)
    L = Kb_ref.shape[1]
    C = 512
    mx = jnp.full((1, S), -jnp.inf, jnp.float32)
    for i in range(L // C):
        stc = jnp.dot(Kb_ref[0, i * C:(i + 1) * C], qrt_scr[...],
                      precision=None) * scale
        st_scr[i * C:(i + 1) * C] = stc
        mx = jnp.maximum(mx, jnp.max(stc, axis=0, keepdims=True))
    den = jnp.zeros((1, S), jnp.float32)
    for i in range(L // C):
        e = jnp.exp(st_scr[i * C:(i + 1) * C] - mx)
        st_scr[i * C:(i + 1) * C] = e
        den = den + jnp.sum(e, axis=0, keepdims=True)
    for i in range(L // C):
        pt_ref[0, i * C:(i + 1) * C] = jnp.dot(
            st_scr[i * C:(i + 1) * C] / den, w_ref[...], precision=None)


def _out_kernel(Vb_ref, pt_ref, WvT_ref, bvr_ref, WoP_ref, bo_ref, out_ref):
    P = pt_ref[0].T
    Ub = jnp.dot(P, Vb_ref[0], precision=None)
    acc = bo_ref[...]
    for h in range(H):
        ctx_h = (jnp.dot(Ub[h:h + 1, :], WvT_ref[:, h * DH:(h + 1) * DH],
                         precision=HI) + bvr_ref[h:h + 1, :])
        acc = acc + jnp.dot(ctx_h, WoP_ref[h], precision=HI)
    out_ref[...] = acc[None]


def kernel(Q, K, V, Wq, bq, Wk, bk, Wv, bv, Wo, bo):
    B, L, _ = Q.shape
    nl = 2
    Lt = L // nl

    idx_s = jax.random.randint(jax.random.key(42), (NTOP,), 0, L)
    Kg = jnp.pad(jnp.take(K, idx_s, axis=1), ((0, 0), (0, NSEL - NTOP), (0, 0)))
    WqT, WkT, WvT = Wq.T, Wk.T, Wv.T
    bvr = bv.reshape(H, DH)
    WoP = Wo.T.reshape(DH, H, DM).transpose(1, 0, 2)
    r = jnp.arange(S)
    w = (((r[:, None] // NSEL) == jnp.arange(H)[None, :])
         & ((r[:, None] % NSEL) < NTOP)).astype(jnp.float32) / NTOP

    Ksp = pl.pallas_call(
        _prep_kernel,
        grid=(B,),
        in_specs=[pl.BlockSpec((1, NSEL, DM), lambda b: (b, 0, 0)),
                  pl.BlockSpec((DM, DM), lambda b: (0, 0)),
                  pl.BlockSpec((1, DM), lambda b: (0, 0))],
        out_specs=pl.BlockSpec((1, NSEL, DM), lambda b: (b, 0, 0)),
        out_shape=jax.ShapeDtypeStruct((B, NSEL, DM), jnp.float32),
    )(Kg, WkT, bk[None])

    Qp, M = pl.pallas_call(
        _score_kernel,
        grid=(B, nl),
        in_specs=[pl.BlockSpec((1, Lt, DM), lambda b, i: (b, i, 0)),
                  pl.BlockSpec((DM, DM), lambda b, i: (0, 0)),
                  pl.BlockSpec((1, DM), lambda b, i: (0, 0)),
                  pl.BlockSpec((1, NSEL, DM), lambda b, i: (b, 0, 0))],
        out_specs=[pl.BlockSpec((1, Lt, DM), lambda b, i: (b, i, 0)),
                   pl.BlockSpec((1, Lt, H), lambda b, i: (b, i, 0))],
        out_shape=[jax.ShapeDtypeStruct((B, L, DM), jnp.float32),
                   jax.ShapeDtypeStruct((B, L, H), jnp.float32)],
    )(Q, WqT, bq[None], Ksp)

    nr = L // 128
    aj = jnp.arange(128)
    UL = (aj[:, None] <= aj[None, :]).astype(jnp.float32)
    ar = jnp.arange(nr)
    UB = (ar[:, None] < ar[None, :]).astype(jnp.float32)

    IDX = pl.pallas_call(
        _thresh_kernel,
        grid=(B,),
        in_specs=[pl.BlockSpec((1, L, H), lambda b: (b, 0, 0)),
                  pl.BlockSpec((128, 128), lambda b: (0, 0)),
                  pl.BlockSpec((nr, nr), lambda b: (0, 0))],
        out_specs=pl.BlockSpec((1, H, NSEL), lambda b: (b, 0, 0)),
        out_shape=jax.ShapeDtypeStruct((B, H, NSEL), jnp.int32),
        scratch_shapes=[pltpu.VMEM((H, nr, 128), jnp.int32)],
    )(M, UL, UB)

    nh2 = H // 2
    qp_flat = Qp.reshape(B * L * nh2, 2 * DH)

    mesh = plsc.VectorSubcoreMesh(core_axis_name="c", subcore_axis_name="s")

    @functools.partial(
        pl.kernel, mesh=mesh,
        out_type=jax.ShapeDtypeStruct((B, H, NSEL, 2 * DH), jnp.float32),
        scratch_types=[pltpu.VMEM((NSEL,), jnp.int32),
                       pltpu.VMEM((NSEL,), jnp.int32),
                       pltpu.VMEM((NSEL, 2 * DH), jnp.float32),
                       pltpu.SemaphoreType.DMA])
    def _sc_gather(idx_hbm, qpflat_hbm, qg_hbm, idx_v, r_v, rows_v, sem):
        wid = lax.axis_index("s") * 2 + lax.axis_index("c")

        @pl.when(wid < B * H)
        def _():
            b = wid // H
            h = wid % H
            pltpu.sync_copy(idx_hbm.at[b, h], idx_v)
            for j in range(NSEL // 16):
                v = idx_v[pl.ds(j * 16, 16)]
                r_v[pl.ds(j * 16, 16)] = v * nh2 + (b * (L * nh2) + h // 2)
            pltpu.async_copy(qpflat_hbm.at[r_v], rows_v, sem).wait()
            pltpu.sync_copy(rows_v, qg_hbm.at[b, h])

    QG = _sc_gather(IDX, qp_flat)

    PT = pl.pallas_call(
        _attn_kernel,
        grid=(B,),
        in_specs=[pl.BlockSpec((1, L, DM), lambda b: (b, 0, 0)),
                  pl.BlockSpec((1, H, NSEL, 2 * DH), lambda b: (b, 0, 0, 0)),
                  pl.BlockSpec((DM, DM), lambda b: (0, 0)),
                  pl.BlockSpec((S, H), lambda b: (0, 0))],
        out_specs=pl.BlockSpec((1, L, H), lambda b: (b, 0, 0)),
        out_shape=jax.ShapeDtypeStruct((B, L, H), jnp.float32),
        scratch_shapes=[pltpu.VMEM((DM, S), jnp.float32),
                        pltpu.VMEM((L, S), jnp.float32)],
    )(K, QG, WkT, w)

    out = pl.pallas_call(
        _out_kernel,
        grid=(B,),
        in_specs=[pl.BlockSpec((1, L, DM), lambda b: (b, 0, 0)),
                  pl.BlockSpec((1, L, H), lambda b: (b, 0, 0)),
                  pl.BlockSpec((DM, DM), lambda b: (0, 0)),
                  pl.BlockSpec((H, DH), lambda b: (0, 0)),
                  pl.BlockSpec((H, DH, DM), lambda b: (0, 0, 0)),
                  pl.BlockSpec((1, DM), lambda b: (0, 0))],
        out_specs=pl.BlockSpec((1, 1, DM), lambda b: (b, 0, 0)),
        out_shape=jax.ShapeDtypeStruct((B, 1, DM), jnp.float32),
    )(V, PT, WvT, bvr, WoP, bo[None])

    return out.reshape(B, DM)


# speedup vs baseline: 1.4894x; 1.1523x over previous
import functools
import math

import jax
import jax.numpy as jnp
from jax import lax
from jax.experimental import pallas as pl
from jax.experimental.pallas import tpu as pltpu
from jax.experimental.pallas import tpu_sc as plsc

H = 12
DH = 64
DM = 768
NTOP = 45
NSEL = 48
S = H * NSEL
HI = lax.Precision.HIGHEST


def _select_kernel(Qb_ref, WqT_ref, bq_ref, Kg_ref, WkT_ref, bk_ref,
                   UL_ref, UB_ref, idx_ref, key_scr):
    L = Qb_ref.shape[1]
    nr = L // 128
    Ksp = jnp.dot(Kg_ref[0], WkT_ref[...], precision=None) + bk_ref[...]
    Qp = jnp.dot(Qb_ref[0], WqT_ref[...], precision=None) + bq_ref[...]
    col = lax.broadcasted_iota(jnp.int32, (L, NSEL), 1)
    cols = []
    for h in range(H):
        QKs_h = lax.dot_general(
            Qp[:, h * DH:(h + 1) * DH], Ksp[:, h * DH:(h + 1) * DH],
            (((1,), (1,)), ((), ())), precision=None)
        mx = jnp.max(jnp.where(col < NTOP, QKs_h, -jnp.inf), axis=1)
        sm = jnp.sum(jnp.where(col < NTOP, QKs_h, 0.0), axis=1)
        cols.append((mx - sm / NTOP)[:, None])
    M = jnp.concatenate(cols, axis=1)

    bits = lax.bitcast_convert_type(M, jnp.int32)
    keys = bits ^ ((bits >> 31) & jnp.int32(0x7FFFFFFF))
    key_scr[...] = keys.T.reshape(H, nr, 128)

    def body(it, lh):
        lo, hi = lh
        mid = (lo >> 1) + (hi >> 1) + (lo & hi & 1)
        cmp = (key_scr[...] > mid).astype(jnp.float32)
        cnt = jnp.sum(jnp.sum(cmp, axis=2, keepdims=True),
                      axis=1, keepdims=True)
        ge = cnt >= float(NTOP)
        return jnp.where(ge, mid, lo), jnp.where(ge, hi, mid)

    lo0 = jnp.full((H, 1, 1), jnp.int32(-2 ** 31))
    hi0 = jnp.full((H, 1, 1), jnp.int32(2 ** 31 - 1))
    _, hi = lax.fori_loop(0, 32, body, (lo0, hi0))

    keys3 = key_scr[...]

    def prefix(msk):
        m = msk.astype(jnp.float32).reshape(H * nr, 128)
        pref = jnp.dot(m, UL_ref[...], precision=None)
        tot = pref[:, 127:128].reshape(H, nr)
        bp = jnp.dot(tot, UB_ref[...], precision=None)
        pos = pref.reshape(H, nr, 128) + bp[:, :, None]
        cg = (bp[:, nr - 1] + tot[:, nr - 1]).reshape(H, 1, 1)
        return pos.astype(jnp.int32), cg.astype(jnp.int32)

    gt = keys3 > hi
    eq = keys3 == hi
    pos_gt, cg = prefix(gt)
    pos_eq, _ = prefix(eq)
    big = jnp.int32(1 << 20)
    slot = jnp.where(gt, pos_gt - 1, jnp.where(eq, cg + pos_eq - 1, big))
    slot = jnp.where(slot < NSEL, slot, big)
    r_io = lax.broadcasted_iota(jnp.int32, (H, nr, 128), 1)
    c_io = lax.broadcasted_iota(jnp.int32, (H, nr, 128), 2)
    lin = (r_io * 128 + c_io).astype(jnp.float32)
    for j in range(NSEL):
        v = jnp.where(slot == j, lin, 0.0)
        red = jnp.sum(jnp.sum(v, axis=2, keepdims=True), axis=1)
        idx_ref[0, :, j:j + 1] = red.astype(jnp.int32)


def _attn_kernel(Kb_ref, QG_ref, WqT_ref, bq_ref, WkT_ref, w_ref, pt_ref,
                 qrt_scr, st_scr):
    scale = 1.0 / math.sqrt(DH)
    for h in range(H):
        Qsel_h = (jnp.dot(QG_ref[0, h], WqT_ref[:, h * DH:(h + 1) * DH],
                          precision=None)
                  + bq_ref[:, h * DH:(h + 1) * DH])
        qrt_scr[:, h * NSEL:(h + 1) * NSEL] = jnp.dot(
            WkT_ref[:, h * DH:(h + 1) * DH], Qsel_h.T, precision=HI)
    L = Kb_ref.shape[1]
    C = 512
    mx = jnp.full((1, S), -jnp.inf, jnp.float32)
    for i in range(L // C):
        stc = jnp.dot(Kb_ref[0, i * C:(i + 1) * C], qrt_scr[...],
                      precision=None) * scale
        st_scr[i * C:(i + 1) * C] = stc
        mx = jnp.maximum(mx, jnp.max(stc, axis=0, keepdims=True))
    den = jnp.zeros((1, S), jnp.float32)
    for i in range(L // C):
        e = jnp.exp(st_scr[i * C:(i + 1) * C] - mx)
        st_scr[i * C:(i + 1) * C] = e
        den = den + jnp.sum(e, axis=0, keepdims=True)
    for i in range(L // C):
        pt_ref[0, i * C:(i + 1) * C] = jnp.dot(
            st_scr[i * C:(i + 1) * C] / den, w_ref[...], precision=None)


def _out_kernel(Vb_ref, pt_ref, WvT_ref, bvr_ref, WoP_ref, bo_ref, out_ref):
    P = pt_ref[0].T
    Ub = jnp.dot(P, Vb_ref[0], precision=None)
    acc = bo_ref[...]
    for h in range(H):
        ctx_h = (jnp.dot(Ub[h:h + 1, :], WvT_ref[:, h * DH:(h + 1) * DH],
                         precision=HI) + bvr_ref[h:h + 1, :])
        acc = acc + jnp.dot(ctx_h, WoP_ref[h], precision=HI)
    out_ref[...] = acc[None]


def kernel(Q, K, V, Wq, bq, Wk, bk, Wv, bv, Wo, bo):
    B, L, _ = Q.shape
    nr = L // 128

    idx_s = jax.random.randint(jax.random.key(42), (NTOP,), 0, L)
    Kg = jnp.pad(jnp.take(K, idx_s, axis=1), ((0, 0), (0, NSEL - NTOP), (0, 0)))
    WqT, WkT, WvT = Wq.T, Wk.T, Wv.T
    bvr = bv.reshape(H, DH)
    WoP = Wo.T.reshape(DH, H, DM).transpose(1, 0, 2)
    r = jnp.arange(S)
    w = (((r[:, None] // NSEL) == jnp.arange(H)[None, :])
         & ((r[:, None] % NSEL) < NTOP)).astype(jnp.float32) / NTOP
    aj = jnp.arange(128)
    UL = (aj[:, None] <= aj[None, :]).astype(jnp.float32)
    ar = jnp.arange(nr)
    UB = (ar[:, None] < ar[None, :]).astype(jnp.float32)

    IDX = pl.pallas_call(
        _select_kernel,
        grid=(B,),
        in_specs=[pl.BlockSpec((1, L, DM), lambda b: (b, 0, 0)),
                  pl.BlockSpec((DM, DM), lambda b: (0, 0)),
                  pl.BlockSpec((1, DM), lambda b: (0, 0)),
                  pl.BlockSpec((1, NSEL, DM), lambda b: (b, 0, 0)),
                  pl.BlockSpec((DM, DM), lambda b: (0, 0)),
                  pl.BlockSpec((1, DM), lambda b: (0, 0)),
                  pl.BlockSpec((128, 128), lambda b: (0, 0)),
                  pl.BlockSpec((nr, nr), lambda b: (0, 0))],
        out_specs=pl.BlockSpec((1, H, NSEL), lambda b: (b, 0, 0)),
        out_shape=jax.ShapeDtypeStruct((B, H, NSEL), jnp.int32),
        scratch_shapes=[pltpu.VMEM((H, nr, 128), jnp.int32)],
    )(Q, WqT, bq[None], Kg, WkT, bk[None], UL, UB)

    q_flat = Q.reshape(B * L, DM)

    mesh = plsc.VectorSubcoreMesh(core_axis_name="c", subcore_axis_name="s")

    @functools.partial(
        pl.kernel, mesh=mesh,
        out_type=jax.ShapeDtypeStruct((B, H, NSEL, DM), jnp.float32),
        scratch_types=[pltpu.VMEM((NSEL,), jnp.int32),
                       pltpu.VMEM((NSEL,), jnp.int32),
                       pltpu.VMEM((NSEL, DM), jnp.float32),
                       pltpu.SemaphoreType.DMA])
    def _sc_gather(idx_hbm, qflat_hbm, qg_hbm, idx_v, r_v, rows_v, sem):
        wid = lax.axis_index("s") * 2 + lax.axis_index("c")

        @pl.when(wid < B * H)
        def _():
            b = wid // H
            h = wid % H
            pltpu.sync_copy(idx_hbm.at[b, h], idx_v)
            for j in range(NSEL // 16):
                v = idx_v[pl.ds(j * 16, 16)]
                r_v[pl.ds(j * 16, 16)] = v + b * L
            pltpu.async_copy(qflat_hbm.at[r_v], rows_v, sem).wait()
            pltpu.sync_copy(rows_v, qg_hbm.at[b, h])

    QG = _sc_gather(IDX, q_flat)

    PT = pl.pallas_call(
        _attn_kernel,
        grid=(B,),
        in_specs=[pl.BlockSpec((1, L, DM), lambda b: (b, 0, 0)),
                  pl.BlockSpec((1, H, NSEL, DM), lambda b: (b, 0, 0, 0)),
                  pl.BlockSpec((DM, DM), lambda b: (0, 0)),
                  pl.BlockSpec((1, DM), lambda b: (0, 0)),
                  pl.BlockSpec((DM, DM), lambda b: (0, 0)),
                  pl.BlockSpec((S, H), lambda b: (0, 0))],
        out_specs=pl.BlockSpec((1, L, H), lambda b: (b, 0, 0)),
        out_shape=jax.ShapeDtypeStruct((B, L, H), jnp.float32),
        scratch_shapes=[pltpu.VMEM((DM, S), jnp.float32),
                        pltpu.VMEM((L, S), jnp.float32)],
    )(K, QG, WqT, bq[None], WkT, w)

    out = pl.pallas_call(
        _out_kernel,
        grid=(B,),
        in_specs=[pl.BlockSpec((1, L, DM), lambda b: (b, 0, 0)),
                  pl.BlockSpec((1, L, H), lambda b: (b, 0, 0)),
                  pl.BlockSpec((DM, DM), lambda b: (0, 0)),
                  pl.BlockSpec((H, DH), lambda b: (0, 0)),
                  pl.BlockSpec((H, DH, DM), lambda b: (0, 0, 0)),
                  pl.BlockSpec((1, DM), lambda b: (0, 0))],
        out_specs=pl.BlockSpec((1, 1, DM), lambda b: (b, 0, 0)),
        out_shape=jax.ShapeDtypeStruct((B, 1, DM), jnp.float32),
    )(V, PT, WvT, bvr, WoP, bo[None])

    return out.reshape(B, DM)
